# Initial kernel scaffold; baseline (speedup 1.0000x reference)
import functools

import jax
import jax.numpy as jnp
from jax import lax
from jax.experimental import pallas as pl
from jax.experimental.pallas import tpu as pltpu
from jax.experimental.pallas import tpu_sc as plsc

N = 10000
M = 20000
E = 320000
D = 128

NC = 2
NS = 16
LL = 16

M2 = M // 2
PADN = N + 16
CH = 512
CG = 256
SB = 4096
EP = NS * 40 * CH
P = EP - E
ET = EP // NS
ET2 = EP // (NC * NS)

_mesh = plsc.VectorSubcoreMesh(
    core_axis_name="c", subcore_axis_name="s", num_cores=NC, num_subcores=NS
)


@functools.partial(
    pl.kernel,
    out_type=(
        jax.ShapeDtypeStruct((N, LL), jnp.float32),
        jax.ShapeDtypeStruct((M, LL), jnp.float32),
    ),
    mesh=_mesh,
    scratch_types=[
        pltpu.VMEM_SHARED((M + 16, LL), jnp.float32),
        pltpu.VMEM((CH,), jnp.int32),
        pltpu.VMEM((CH, LL), jnp.float32),
    ],
)
def _hist_kernel(srcA, dstA, onesrows, zhist, degn, degm, hist, ibuf, obuf):
    c = lax.axis_index("c")
    s = lax.axis_index("s")
    rz = (M + 16) // NS
    pltpu.sync_copy(zhist.at[pl.ds(s * rz, rz)], hist.at[pl.ds(s * rz, rz)])
    pltpu.sync_copy(onesrows, obuf)
    plsc.subcore_barrier()

    def run(idx_hbm):
        def body(j, carry):
            pltpu.sync_copy(idx_hbm.at[pl.ds(s * ET + j * CH, CH)], ibuf)
            pltpu.sync_copy(obuf, hist.at[ibuf], add=True)
            return carry

        lax.fori_loop(0, ET // CH, body, 0)

    @pl.when(c == 0)
    def _():
        run(srcA)

    @pl.when(c == 1)
    def _():
        run(dstA)

    plsc.subcore_barrier()

    @pl.when(c == 0)
    def _():
        r = N // NS
        pltpu.sync_copy(hist.at[pl.ds(s * r, r)], degn.at[pl.ds(s * r, r)])

    @pl.when(c == 1)
    def _():
        r = M // NS
        pltpu.sync_copy(hist.at[pl.ds(s * r, r)], degm.at[pl.ds(s * r, r)])


@functools.partial(
    pl.kernel,
    out_type=jax.ShapeDtypeStruct((M, D), jnp.float32),
    mesh=_mesh,
    scratch_types=[
        pltpu.VMEM_SHARED((PADN, D), jnp.float32),
        pltpu.VMEM((SB,), jnp.int32),
        pltpu.VMEM((SB,), jnp.int32),
        pltpu.VMEM((ET + CG,), jnp.int32),
        pltpu.VMEM((ET // CG + 1, CG), jnp.int32),
        pltpu.VMEM((CG, D), jnp.float32),
        pltpu.SemaphoreType.DMA,
    ],
)
def _scatter1_kernel(x_hbm, srcA, dstA, zacc, out, acc, ssrc, sdst, csrc, cdst,
                     rows, sem):
    c = lax.axis_index("c")
    s = lax.axis_index("s")
    rz = PADN // NS
    pltpu.sync_copy(zacc.at[pl.ds(s * rz, rz)], acc.at[pl.ds(s * rz, rz)])
    plsc.subcore_barrier()

    lo = c * M2
    iota = lax.iota(jnp.int32, LL)

    def outer(b, count):
        base = s * ET + b * SB
        pltpu.sync_copy(srcA.at[pl.ds(base, SB)], ssrc)
        pltpu.sync_copy(dstA.at[pl.ds(base, SB)], sdst)

        def inner(i, cnt):
            dvec = sdst[pl.ds(i * LL, LL)]
            svec = ssrc[pl.ds(i * LL, LL)]
            ld = dvec - lo
            mask = (ld >= 0) & (ld < M2)
            mi = jnp.where(mask, 1, 0).astype(jnp.int32)
            offs = cnt + plsc.cumsum(mi) - 1
            plsc.store_scatter(csrc, [offs], svec, mask=mask)
            plsc.store_scatter(
                cdst,
                [lax.shift_right_logical(offs, 8), offs & (CG - 1)],
                ld,
                mask=mask,
            )
            return cnt + jnp.sum(mi)

        return lax.fori_loop(0, SB // LL, inner, count)

    K = lax.fori_loop(0, ET // SB, outer, jnp.int32(0))

    dump = jnp.full((LL,), M2, jnp.int32) + s
    for i in range(CG // LL):
        offs = K + i * LL + iota
        plsc.store_scatter(csrc, [offs], iota)
        plsc.store_scatter(
            cdst, [lax.shift_right_logical(offs, 8), offs & (CG - 1)], dump
        )

    nch = (K + CG - 1) // CG

    def gs(j, carry):
        pltpu.async_copy(x_hbm.at[csrc.at[pl.ds(j * CG, CG)]], rows, sem).wait()
        pltpu.sync_copy(rows, acc.at[cdst.at[j]], add=True)
        return carry

    lax.fori_loop(0, nch, gs, 0)
    plsc.subcore_barrier()

    r = M2 // NS
    pltpu.sync_copy(acc.at[pl.ds(s * r, r)], out.at[pl.ds(c * M2 + s * r, r)])


@functools.partial(
    pl.kernel,
    out_type=jax.ShapeDtypeStruct((NC, N, D), jnp.float32),
    mesh=_mesh,
    scratch_types=[
        pltpu.VMEM_SHARED((PADN, D), jnp.float32),
        pltpu.VMEM((CG,), jnp.int32),
        pltpu.VMEM((CG,), jnp.int32),
        pltpu.VMEM((CG, D), jnp.float32),
        pltpu.SemaphoreType.DMA,
    ],
)
def _scatter2_kernel(y_hbm, dstC, srcA, zacc, out, acc, gbuf, sbuf, rows, sem):
    c = lax.axis_index("c")
    s = lax.axis_index("s")
    rz = PADN // NS
    pltpu.sync_copy(zacc.at[pl.ds(s * rz, rz)], acc.at[pl.ds(s * rz, rz)])
    plsc.subcore_barrier()

    def gs(j, carry):
        base = c * (EP // NC) + s * ET2 + j * CG
        pltpu.sync_copy(dstC.at[pl.ds(base, CG)], gbuf)
        pltpu.sync_copy(srcA.at[pl.ds(base, CG)], sbuf)
        pltpu.async_copy(y_hbm.at[gbuf], rows, sem).wait()
        pltpu.sync_copy(rows, acc.at[sbuf], add=True)
        return carry

    lax.fori_loop(0, ET2 // CG, gs, 0)
    plsc.subcore_barrier()

    r = N // NS
    pltpu.sync_copy(acc.at[pl.ds(s * r, r)], out.at[c, pl.ds(s * r, r)])


_BN = 400


def _k1_body(h_ref, w1_ref, degn_ref, x_ref):
    nrm = lax.rsqrt(jnp.maximum(degn_ref[:, 0:1], 1.0))
    x_ref[...] = (
        jnp.dot(h_ref[...], w1_ref[...], preferred_element_type=jnp.float32)
        * nrm
    )


def _k1(h, W1, degn):
    return pl.pallas_call(
        _k1_body,
        out_shape=jax.ShapeDtypeStruct((N, D), jnp.float32),
        grid=(N // _BN,),
        in_specs=[
            pl.BlockSpec((_BN, D), lambda i: (i, 0)),
            pl.BlockSpec((D, D), lambda i: (0, 0)),
            pl.BlockSpec((_BN, LL), lambda i: (i, 0)),
        ],
        out_specs=pl.BlockSpec((_BN, D), lambda i: (i, 0)),
    )(h, W1, degn)


def _kmid_body(a_ref, degm_ref, w2_ref, b1_ref, y_ref):
    nrm = lax.rsqrt(jnp.maximum(degm_ref[:, 0:1], 1.0))
    t = (a_ref[...] * nrm + b1_ref[...]) * nrm
    y_ref[...] = jnp.dot(t, w2_ref[...], preferred_element_type=jnp.float32)


def _kmid(accm, degm, W2, b1):
    return pl.pallas_call(
        _kmid_body,
        out_shape=jax.ShapeDtypeStruct((M, D), jnp.float32),
        grid=(M // _BN,),
        in_specs=[
            pl.BlockSpec((_BN, D), lambda i: (i, 0)),
            pl.BlockSpec((_BN, LL), lambda i: (i, 0)),
            pl.BlockSpec((D, D), lambda i: (0, 0)),
            pl.BlockSpec((1, D), lambda i: (0, 0)),
        ],
        out_specs=pl.BlockSpec((_BN, D), lambda i: (i, 0)),
    )(accm, degm, W2, b1)


def _kpost_body(h_ref, a0_ref, a1_ref, degn_ref, b2_ref, w3_ref, b3_ref,
                w4_ref, b4_ref, g1_ref, be1_ref, g2_ref, be2_ref, o_ref):
    nrm = lax.rsqrt(jnp.maximum(degn_ref[:, 0:1], 1.0))
    h_node = (a0_ref[...] + a1_ref[...]) * nrm + b2_ref[...]
    t = h_ref[...] + h_node
    mu = jnp.mean(t, axis=-1, keepdims=True)
    var = jnp.mean((t - mu) ** 2, axis=-1, keepdims=True)
    h1 = (t - mu) * lax.rsqrt(var + 1e-5) * g1_ref[...] + be1_ref[...]
    f = jnp.maximum(
        jnp.dot(h1, w3_ref[...], preferred_element_type=jnp.float32)
        + b3_ref[...],
        0.0,
    )
    hf = (
        jnp.dot(f, w4_ref[...], preferred_element_type=jnp.float32)
        + b4_ref[...]
    )
    t2 = h1 + hf
    mu2 = jnp.mean(t2, axis=-1, keepdims=True)
    var2 = jnp.mean((t2 - mu2) ** 2, axis=-1, keepdims=True)
    o_ref[...] = (
        (t2 - mu2) * lax.rsqrt(var2 + 1e-5) * g2_ref[...] + be2_ref[...]
    )


def _kpost(h, a0, a1, degn, b2, W3, b3, W4, b4, g1, be1, g2, be2):
    row = lambda i: (i, 0)
    fix2 = lambda i: (0, 0)
    return pl.pallas_call(
        _kpost_body,
        out_shape=jax.ShapeDtypeStruct((N, D), jnp.float32),
        grid=(N // _BN,),
        in_specs=[
            pl.BlockSpec((_BN, D), row),
            pl.BlockSpec((_BN, D), row),
            pl.BlockSpec((_BN, D), row),
            pl.BlockSpec((_BN, LL), row),
            pl.BlockSpec((1, D), fix2),
            pl.BlockSpec((D, 4 * D), fix2),
            pl.BlockSpec((1, 4 * D), fix2),
            pl.BlockSpec((4 * D, D), fix2),
            pl.BlockSpec((1, D), fix2),
            pl.BlockSpec((1, D), fix2),
            pl.BlockSpec((1, D), fix2),
            pl.BlockSpec((1, D), fix2),
            pl.BlockSpec((1, D), fix2),
        ],
        out_specs=pl.BlockSpec((_BN, D), row),
    )(h, a0, a1, degn, b2, W3, b3, W4, b4, g1, be1, g2, be2)


def kernel(h, src_idx, dst_idx, W1, b1, W2, b2, W3, b3, W4, b4, g1, be1, g2,
           be2):
    pad = (jnp.arange(P, dtype=jnp.int32) % LL).astype(jnp.int32)
    srcA = jnp.concatenate([src_idx, N + pad])
    dstA = jnp.concatenate([dst_idx, M + pad])
    dstC = jnp.concatenate([dst_idx, pad])
    onesrows = jnp.zeros((CH, LL), jnp.float32).at[:, 0].set(1.0)
    zacc = jnp.zeros((PADN, D), jnp.float32)
    zhist = jnp.zeros((M + 16, LL), jnp.float32)

    degn, degm = _hist_kernel(srcA, dstA, onesrows, zhist)
    x = _k1(h, W1, degn)
    accm = _scatter1_kernel(x, srcA, dstA, zacc)
    y = _kmid(accm, degm, W2, b1.reshape(1, D))
    accn = _scatter2_kernel(y, dstC, srcA, zacc)
    out = _kpost(
        h, accn[0], accn[1], degn, b2.reshape(1, D), W3, b3.reshape(1, 4 * D),
        W4, b4.reshape(1, D), g1.reshape(1, D), be1.reshape(1, D),
        g2.reshape(1, D), be2.reshape(1, D),
    )
    return out


# speedup vs baseline: 4.6943x; 4.6943x over previous
import functools

import jax
import jax.numpy as jnp
from jax import lax
from jax.experimental import pallas as pl
from jax.experimental.pallas import tpu as pltpu
from jax.experimental.pallas import tpu_sc as plsc

N = 10000
M = 20000
E = 320000
D = 128

NC = 2
NS = 16
LL = 16

RS = 5000
AQ = 5120
CG = 128
SB = 4096
EP = NS * 160 * CG
P = EP - E
ET = EP // NS

_mesh = plsc.VectorSubcoreMesh(
    core_axis_name="c", subcore_axis_name="s", num_cores=NC, num_subcores=NS
)

_RZ = AQ // NS
_ZCH = ((0, 128), (128, 128), (256, 64))


def _make_scatter(npass, gather):
    scratch = [
        pltpu.VMEM_SHARED((AQ, D), jnp.float32),
        pltpu.VMEM((SB,), jnp.int32),
        pltpu.VMEM((ET + CG,), jnp.int32),
        pltpu.VMEM((CG, D), jnp.float32),
        pltpu.VMEM((CG, D), jnp.float32),
        pltpu.SemaphoreType.DMA,
    ]
    if gather:
        scratch[2:2] = [
            pltpu.VMEM((SB,), jnp.int32),
            pltpu.VMEM((ET + CG,), jnp.int32),
        ]

    def _body(tab, gidx, fidx, onesr, zrows, out, acc, sf, sg, cg_, cd,
              rows, zbuf, sem):
        c = lax.axis_index("c")
        s = lax.axis_index("s")
        iota = lax.iota(jnp.int32, LL)
        dump = jnp.full((LL,), RS, jnp.int32) + s

        pltpu.sync_copy(zrows, zbuf)
        if not gather:
            pltpu.sync_copy(onesr, rows)

        for p in range(npass):
            lo = (c * npass + p) * RS

            for off, sz in _ZCH:
                pltpu.sync_copy(zbuf.at[pl.ds(0, sz)],
                                acc.at[pl.ds(s * _RZ + off, sz)])
            plsc.subcore_barrier()

            def outer(b, count):
                base = s * ET + b * SB
                pltpu.sync_copy(fidx.at[pl.ds(base, SB)], sf)
                if gather:
                    pltpu.sync_copy(gidx.at[pl.ds(base, SB)], sg)

                def inner(i, cnt):
                    fvec = sf[pl.ds(i * LL, LL)]
                    ld = fvec - lo
                    mask = (ld >= 0) & (ld < RS)
                    mi = jnp.where(mask, 1, 0).astype(jnp.int32)
                    offs = cnt + plsc.cumsum(mi) - 1
                    plsc.store_scatter(cd, [offs], ld, mask=mask)
                    if gather:
                        gvec = sg[pl.ds(i * LL, LL)]
                        plsc.store_scatter(cg_, [offs], gvec, mask=mask)
                    return cnt + jnp.sum(mi)

                return lax.fori_loop(0, SB // LL, inner, count)

            K = lax.fori_loop(0, ET // SB, outer, jnp.int32(0))

            for i in range(CG // LL):
                offs = K + i * LL + iota
                plsc.store_scatter(cd, [offs], dump)
                if gather:
                    plsc.store_scatter(cg_, [offs], iota)

            nch = (K + CG - 1) // CG

            def gs(j, carry):
                if gather:
                    pltpu.async_copy(
                        tab.at[cg_.at[pl.ds(j * CG, CG)]], rows, sem
                    ).wait()
                pltpu.sync_copy(rows, acc.at[cd.at[pl.ds(j * CG, CG)]],
                                add=True)
                return carry

            lax.fori_loop(0, nch, gs, 0)
            plsc.subcore_barrier()

            for off, sz in _ZCH:
                pltpu.sync_copy(acc.at[pl.ds(s * _RZ + off, sz)],
                                zbuf.at[pl.ds(0, sz)])
                pltpu.sync_copy(zbuf.at[pl.ds(0, sz)],
                                out.at[c * npass + p,
                                       pl.ds(s * _RZ + off, sz)])
            if p + 1 < npass:
                pltpu.sync_copy(zrows, zbuf)
                plsc.subcore_barrier()

    out_type = jax.ShapeDtypeStruct((NC * npass, AQ, D), jnp.float32)
    cp = pltpu.CompilerParams(needs_layout_passes=False)

    if gather:
        @functools.partial(pl.kernel, out_type=out_type, mesh=_mesh,
                           compiler_params=cp, scratch_types=scratch)
        def _k(tab, gidx, fidx, zrows, out, acc, sf, sg, cg_, cd, rows,
               zbuf, sem):
            _body(tab, gidx, fidx, None, zrows, out, acc, sf, sg, cg_, cd,
                  rows, zbuf, sem)
    else:
        @functools.partial(pl.kernel, out_type=out_type, mesh=_mesh,
                           compiler_params=cp, scratch_types=scratch)
        def _k(onesr, zrows, fidx, out, acc, sf, cd, rows, zbuf, sem):
            _body(None, None, fidx, onesr, zrows, out, acc, sf, None, None,
                  cd, rows, zbuf, sem)

    return _k


_histn_kernel = _make_scatter(1, gather=False)
_histm_kernel = _make_scatter(2, gather=False)
_scatter1_kernel = _make_scatter(2, gather=True)
_scatter2_kernel = _make_scatter(1, gather=True)


_BS = 200
_NB = RS // _BS


def _deg_spec():
    return pl.BlockSpec((1, _BS, D), lambda i: (i // _NB, i % _NB, 0))


def _k1_body(h_ref, w1_ref, degn_ref, x_ref):
    nrm = lax.rsqrt(jnp.maximum(degn_ref[0][:, 0:1], 1.0))
    x_ref[...] = (
        jnp.dot(h_ref[...], w1_ref[...], preferred_element_type=jnp.float32)
        * nrm
    )


def _k1(h, W1, degn):
    return pl.pallas_call(
        _k1_body,
        out_shape=jax.ShapeDtypeStruct((N, D), jnp.float32),
        grid=(N // _BS,),
        in_specs=[
            pl.BlockSpec((_BS, D), lambda i: (i, 0)),
            pl.BlockSpec((D, D), lambda i: (0, 0)),
            _deg_spec(),
        ],
        out_specs=pl.BlockSpec((_BS, D), lambda i: (i, 0)),
    )(h, W1, degn)


def _kmid_body(a_ref, degm_ref, w2_ref, b1_ref, y_ref):
    nrm = lax.rsqrt(jnp.maximum(degm_ref[0][:, 0:1], 1.0))
    t = (a_ref[0] * nrm + b1_ref[...]) * nrm
    y_ref[...] = jnp.dot(t, w2_ref[...], preferred_element_type=jnp.float32)


def _kmid(accm, degm, W2, b1):
    return pl.pallas_call(
        _kmid_body,
        out_shape=jax.ShapeDtypeStruct((M, D), jnp.float32),
        grid=(M // _BS,),
        in_specs=[
            _deg_spec(),
            _deg_spec(),
            pl.BlockSpec((D, D), lambda i: (0, 0)),
            pl.BlockSpec((1, D), lambda i: (0, 0)),
        ],
        out_specs=pl.BlockSpec((_BS, D), lambda i: (i, 0)),
    )(accm, degm, W2, b1)


def _kpost_body(h_ref, a_ref, degn_ref, b2_ref, w3_ref, b3_ref,
                w4_ref, b4_ref, g1_ref, be1_ref, g2_ref, be2_ref, o_ref):
    nrm = lax.rsqrt(jnp.maximum(degn_ref[0][:, 0:1], 1.0))
    h_node = a_ref[0] * nrm + b2_ref[...]
    t = h_ref[...] + h_node
    mu = jnp.mean(t, axis=-1, keepdims=True)
    var = jnp.mean((t - mu) ** 2, axis=-1, keepdims=True)
    h1 = (t - mu) * lax.rsqrt(var + 1e-5) * g1_ref[...] + be1_ref[...]
    f = jnp.maximum(
        jnp.dot(h1, w3_ref[...], preferred_element_type=jnp.float32)
        + b3_ref[...],
        0.0,
    )
    hf = (
        jnp.dot(f, w4_ref[...], preferred_element_type=jnp.float32)
        + b4_ref[...]
    )
    t2 = h1 + hf
    mu2 = jnp.mean(t2, axis=-1, keepdims=True)
    var2 = jnp.mean((t2 - mu2) ** 2, axis=-1, keepdims=True)
    o_ref[...] = (
        (t2 - mu2) * lax.rsqrt(var2 + 1e-5) * g2_ref[...] + be2_ref[...]
    )


def _kpost(h, accn, degn, b2, W3, b3, W4, b4, g1, be1, g2, be2):
    row = lambda i: (i, 0)
    fix2 = lambda i: (0, 0)
    return pl.pallas_call(
        _kpost_body,
        out_shape=jax.ShapeDtypeStruct((N, D), jnp.float32),
        grid=(N // _BS,),
        in_specs=[
            pl.BlockSpec((_BS, D), row),
            _deg_spec(),
            _deg_spec(),
            pl.BlockSpec((1, D), fix2),
            pl.BlockSpec((D, 4 * D), fix2),
            pl.BlockSpec((1, 4 * D), fix2),
            pl.BlockSpec((4 * D, D), fix2),
            pl.BlockSpec((1, D), fix2),
            pl.BlockSpec((1, D), fix2),
            pl.BlockSpec((1, D), fix2),
            pl.BlockSpec((1, D), fix2),
            pl.BlockSpec((1, D), fix2),
        ],
        out_specs=pl.BlockSpec((_BS, D), row),
    )(h, accn, degn, b2, W3, b3, W4, b4, g1, be1, g2, be2)


def kernel(h, src_idx, dst_idx, W1, b1, W2, b2, W3, b3, W4, b4, g1, be1, g2,
           be2):
    pad = (jnp.arange(P, dtype=jnp.int32) % LL).astype(jnp.int32)
    srcA = jnp.concatenate([src_idx, N + pad])
    dstA = jnp.concatenate([dst_idx, M + pad])
    dstC = jnp.concatenate([dst_idx, pad])
    onesr = jnp.ones((CG, D), jnp.float32)
    zrows = jnp.zeros((CG, D), jnp.float32)

    degn = _histn_kernel(onesr, zrows, srcA)
    degm = _histm_kernel(onesr, zrows, dstA)
    x = _k1(h, W1, degn)
    accm = _scatter1_kernel(x, srcA, dstA, zrows)
    y = _kmid(accm, degm, W2, b1.reshape(1, D))
    accn = _scatter2_kernel(y, dstC, srcA, zrows)
    out = _kpost(
        h, accn, degn, b2.reshape(1, D), W3, b3.reshape(1, 4 * D),
        W4, b4.reshape(1, D), g1.reshape(1, D), be1.reshape(1, D),
        g2.reshape(1, D), be2.reshape(1, D),
    )
    return out


# speedup vs baseline: 5.4859x; 1.1686x over previous
import functools

import jax
import jax.numpy as jnp
from jax import lax
from jax.experimental import pallas as pl
from jax.experimental.pallas import tpu as pltpu
from jax.experimental.pallas import tpu_sc as plsc

N = 10000
M = 20000
E = 320000
D = 128

NC = 2
NS = 16
LL = 16

RS = 5000
AQ = 5120
CG = 128
SB = 4096
EP = NS * 160 * CG
P = EP - E
ET = EP // NS

_mesh = plsc.VectorSubcoreMesh(
    core_axis_name="c", subcore_axis_name="s", num_cores=NC, num_subcores=NS
)

_RZ = AQ // NS
_ZCH = ((0, 128), (128, 128), (256, 64))


def _make_scatter(npass, gather):
    W = D
    scratch = [
        pltpu.VMEM_SHARED((AQ, W), jnp.float32),
        pltpu.VMEM((SB,), jnp.int32),
        pltpu.VMEM((ET + CG,), jnp.int32),
        pltpu.VMEM((CG, W), jnp.float32),
        pltpu.VMEM((CG, W), jnp.float32),
        pltpu.SemaphoreType.DMA,
        pltpu.SemaphoreType.DMA,
    ]
    if gather:
        scratch[2:2] = [
            pltpu.VMEM((SB,), jnp.int32),
            pltpu.VMEM((ET + CG,), jnp.int32),
        ]

    def _body(tab, gidx, fidx, onesr, zrows, out, acc, sf, sg, cg_, cd,
              rows, zbuf, sem, semb):
        c = lax.axis_index("c")
        s = lax.axis_index("s")
        iota = lax.iota(jnp.int32, LL)
        dump = jnp.full((LL,), RS, jnp.int32) + s

        if not gather:
            pltpu.sync_copy(onesr, rows)

        for p in range(npass):
            lo = (c * npass + p) * RS

            pltpu.sync_copy(zrows, zbuf)
            for off, sz in _ZCH:
                pltpu.sync_copy(zbuf.at[pl.ds(0, sz)],
                                acc.at[pl.ds(s * _RZ + off, sz)])
            plsc.subcore_barrier()

            def outer(b, count):
                base = s * ET + b * SB
                pltpu.sync_copy(fidx.at[pl.ds(base, SB)], sf)
                if gather:
                    pltpu.sync_copy(gidx.at[pl.ds(base, SB)], sg)

                def inner(i, cnt):
                    fvec = sf[pl.ds(i * LL, LL)]
                    ld = fvec - lo
                    mask = (ld >= 0) & (ld < RS)
                    mi = jnp.where(mask, 1, 0).astype(jnp.int32)
                    offs = cnt + plsc.cumsum(mi) - 1
                    plsc.store_scatter(cd, [offs], ld, mask=mask)
                    if gather:
                        gvec = sg[pl.ds(i * LL, LL)]
                        plsc.store_scatter(cg_, [offs], gvec, mask=mask)
                    return cnt + jnp.sum(mi)

                return lax.fori_loop(0, SB // LL, inner, count)

            K = lax.fori_loop(0, ET // SB, outer, jnp.int32(0))

            for i in range(CG // LL):
                offs = K + i * LL + iota
                plsc.store_scatter(cd, [offs], dump)
                if gather:
                    plsc.store_scatter(cg_, [offs], iota)

            nch = (K + CG - 1) // CG

            if gather:
                gmax = jnp.maximum(nch - 1, 0)

                def _issue(g, buf, sm):
                    base = jnp.minimum(g, gmax) * CG
                    pltpu.async_copy(tab.at[cg_.at[pl.ds(base, CG)]],
                                     buf, sm)

                def _wait(buf, sm):
                    pltpu.make_async_copy(
                        tab.at[cg_.at[pl.ds(0, CG)]], buf, sm).wait()

                def _scat(g, buf):
                    pltpu.sync_copy(buf, acc.at[cd.at[pl.ds(g * CG, CG)]],
                                    add=True)

                _issue(jnp.int32(0), rows, sem)

                def gs2(jj, carry):
                    g = jj * 2
                    _wait(rows, sem)
                    _issue(g + 1, zbuf, semb)
                    _scat(g, rows)
                    _wait(zbuf, semb)
                    _issue(g + 2, rows, sem)
                    _scat(g + 1, zbuf)
                    return carry

                lax.fori_loop(0, nch // 2, gs2, 0)
                _wait(rows, sem)

                @pl.when(nch % 2 == 1)
                def _():
                    _scat(nch - 1, rows)
            else:
                def gs(j, carry):
                    pltpu.sync_copy(rows,
                                    acc.at[cd.at[pl.ds(j * CG, CG)]],
                                    add=True)
                    return carry

                lax.fori_loop(0, nch, gs, 0)
            plsc.subcore_barrier()

            for off, sz in _ZCH:
                pltpu.sync_copy(acc.at[pl.ds(s * _RZ + off, sz)],
                                zbuf.at[pl.ds(0, sz)])
                pltpu.sync_copy(zbuf.at[pl.ds(0, sz)],
                                out.at[c * npass + p,
                                       pl.ds(s * _RZ + off, sz)])
            if p + 1 < npass:
                plsc.subcore_barrier()

    out_type = jax.ShapeDtypeStruct((NC * npass, AQ, W), jnp.float32)
    cp = pltpu.CompilerParams(needs_layout_passes=False)

    if gather:
        @functools.partial(pl.kernel, out_type=out_type, mesh=_mesh,
                           compiler_params=cp, scratch_types=scratch)
        def _k(tab, gidx, fidx, zrows, out, acc, sf, sg, cg_, cd, rows,
               zbuf, sem, semb):
            _body(tab, gidx, fidx, None, zrows, out, acc, sf, sg, cg_, cd,
                  rows, zbuf, sem, semb)
    else:
        @functools.partial(pl.kernel, out_type=out_type, mesh=_mesh,
                           compiler_params=cp, scratch_types=scratch)
        def _k(onesr, zrows, fidx, out, acc, sf, cd, rows, zbuf, sem, semb):
            _body(None, None, fidx, onesr, zrows, out, acc, sf, None, None,
                  cd, rows, zbuf, sem, semb)

    return _k


_histn_kernel = _make_scatter(1, gather=False)
_histm_kernel = _make_scatter(2, gather=False)
_scatter1_kernel = _make_scatter(2, gather=True)
_scatter2_kernel = _make_scatter(1, gather=True)


_BS = 200
_NB = RS // _BS


def _deg_spec():
    return pl.BlockSpec((1, _BS, D), lambda i: (i // _NB, i % _NB, 0))


def _k1_body(h_ref, w1_ref, degn_ref, x_ref):
    nrm = lax.rsqrt(jnp.maximum(degn_ref[0][:, 0:1], 1.0))
    x_ref[...] = (
        jnp.dot(h_ref[...], w1_ref[...], preferred_element_type=jnp.float32)
        * nrm
    )


def _k1(h, W1, degn):
    return pl.pallas_call(
        _k1_body,
        out_shape=jax.ShapeDtypeStruct((N, D), jnp.float32),
        grid=(N // _BS,),
        in_specs=[
            pl.BlockSpec((_BS, D), lambda i: (i, 0)),
            pl.BlockSpec((D, D), lambda i: (0, 0)),
            _deg_spec(),
        ],
        out_specs=pl.BlockSpec((_BS, D), lambda i: (i, 0)),
    )(h, W1, degn)


def _kmid_body(a_ref, degm_ref, w2_ref, b1_ref, y_ref):
    nrm = lax.rsqrt(jnp.maximum(degm_ref[0][:, 0:1], 1.0))
    t = (a_ref[0] * nrm + b1_ref[...]) * nrm
    y_ref[...] = jnp.dot(t, w2_ref[...], preferred_element_type=jnp.float32)


def _kmid(accm, degm, W2, b1):
    return pl.pallas_call(
        _kmid_body,
        out_shape=jax.ShapeDtypeStruct((M, D), jnp.float32),
        grid=(M // _BS,),
        in_specs=[
            pl.BlockSpec((1, _BS, D), lambda i: (i // _NB, i % _NB, 0)),
            _deg_spec(),
            pl.BlockSpec((D, D), lambda i: (0, 0)),
            pl.BlockSpec((1, D), lambda i: (0, 0)),
        ],
        out_specs=pl.BlockSpec((_BS, D), lambda i: (i, 0)),
    )(accm, degm, W2, b1)


def _kpost_body(h_ref, a_ref, degn_ref, b2_ref, w3_ref, b3_ref,
                w4_ref, b4_ref, g1_ref, be1_ref, g2_ref, be2_ref, o_ref):
    nrm = lax.rsqrt(jnp.maximum(degn_ref[0][:, 0:1], 1.0))
    h_node = a_ref[0] * nrm + b2_ref[...]
    t = h_ref[...] + h_node
    mu = jnp.mean(t, axis=-1, keepdims=True)
    var = jnp.mean((t - mu) ** 2, axis=-1, keepdims=True)
    h1 = (t - mu) * lax.rsqrt(var + 1e-5) * g1_ref[...] + be1_ref[...]
    f = jnp.maximum(
        jnp.dot(h1, w3_ref[...], preferred_element_type=jnp.float32)
        + b3_ref[...],
        0.0,
    )
    hf = (
        jnp.dot(f, w4_ref[...], preferred_element_type=jnp.float32)
        + b4_ref[...]
    )
    t2 = h1 + hf
    mu2 = jnp.mean(t2, axis=-1, keepdims=True)
    var2 = jnp.mean((t2 - mu2) ** 2, axis=-1, keepdims=True)
    o_ref[...] = (
        (t2 - mu2) * lax.rsqrt(var2 + 1e-5) * g2_ref[...] + be2_ref[...]
    )


def _kpost(h, accn, degn, b2, W3, b3, W4, b4, g1, be1, g2, be2):
    row = lambda i: (i, 0)
    fix2 = lambda i: (0, 0)
    return pl.pallas_call(
        _kpost_body,
        out_shape=jax.ShapeDtypeStruct((N, D), jnp.float32),
        grid=(N // _BS,),
        in_specs=[
            pl.BlockSpec((_BS, D), row),
            pl.BlockSpec((1, _BS, D), lambda i: (i // _NB, i % _NB, 0)),
            _deg_spec(),
            pl.BlockSpec((1, D), fix2),
            pl.BlockSpec((D, 4 * D), fix2),
            pl.BlockSpec((1, 4 * D), fix2),
            pl.BlockSpec((4 * D, D), fix2),
            pl.BlockSpec((1, D), fix2),
            pl.BlockSpec((1, D), fix2),
            pl.BlockSpec((1, D), fix2),
            pl.BlockSpec((1, D), fix2),
            pl.BlockSpec((1, D), fix2),
        ],
        out_specs=pl.BlockSpec((_BS, D), row),
    )(h, accn, degn, b2, W3, b3, W4, b4, g1, be1, g2, be2)


def kernel(h, src_idx, dst_idx, W1, b1, W2, b2, W3, b3, W4, b4, g1, be1, g2,
           be2):
    pad = (jnp.arange(P, dtype=jnp.int32) % LL).astype(jnp.int32)
    srcA = jnp.concatenate([src_idx, N + pad])
    dstA = jnp.concatenate([dst_idx, M + pad])
    dstC = jnp.concatenate([dst_idx, pad])
    onesr = jnp.ones((CG, D), jnp.float32)
    zrows = jnp.zeros((CG, D), jnp.float32)

    degn = _histn_kernel(onesr, zrows, srcA)
    degm = _histm_kernel(onesr, zrows, dstA)
    x = _k1(h, W1, degn)
    accm = _scatter1_kernel(x, srcA, dstA, zrows)
    y = _kmid(accm, degm, W2, b1.reshape(1, D))
    accn = _scatter2_kernel(y, dstC, srcA, zrows)
    out = _kpost(
        h, accn, degn, b2.reshape(1, D), W3, b3.reshape(1, 4 * D),
        W4, b4.reshape(1, D), g1.reshape(1, D), be1.reshape(1, D),
        g2.reshape(1, D), be2.reshape(1, D),
    )
    return out
